# baseline (speedup 1.0000x reference)
import jax
import jax.numpy as jnp
from jax import lax
from jax.experimental import pallas as pl
from jax.experimental.pallas import tpu as pltpu
from jax.experimental.pallas import tpu_sc as plsc

N = 10000
D = 128
H = D // 2
E = 320000
ALPHA = 0.1
CK = 0.45

NC = 2
NS = 16
C = 128
KB = 2
KR = 2
NCH = 160
NPH = NCH // KR
E_PAD = NS * NCH * C
N_PAD = 10240
RPT = N_PAD // NS
TPC = RPT // C

_f32 = jnp.float32
_mesh = plsc.VectorSubcoreMesh(core_axis_name="c", subcore_axis_name="s")
_sc_params = pltpu.CompilerParams(use_tc_tiling_on_sc=False)


def _hist_body(coli_hbm, z16_hbm, out_hbm, colidx_v, ones_v, acc_sh, *sems):
    c = lax.axis_index("c")
    s = lax.axis_index("s")
    w = c * NS + s
    pltpu.sync_copy(coli_hbm.at[w], colidx_v)

    def fill(i, carry):
        ones_v[i, :] = jnp.ones((16,), _f32)
        return carry

    lax.fori_loop(0, C, fill, 0)
    pltpu.sync_copy(z16_hbm.at[pl.ds(s * RPT, RPT)],
                    acc_sh.at[pl.ds(s * RPT, RPT)])
    plsc.subcore_barrier()

    def phase(p, carry):
        descs = [
            pltpu.async_copy(ones_v, acc_sh.at[colidx_v.at[p * KB + j]],
                             sems[j], add=True)
            for j in range(KB)
        ]
        for d in descs:
            d.wait()
        return carry

    lax.fori_loop(0, (NCH // NC) // KB, phase, 0)
    plsc.subcore_barrier()
    pltpu.sync_copy(acc_sh.at[pl.ds(s * RPT, RPT)],
                    out_hbm.at[c, pl.ds(s * RPT, RPT)])


_hist_kernel = pl.kernel(
    _hist_body,
    out_type=jax.ShapeDtypeStruct((NC, N_PAD, 16), _f32),
    mesh=_mesh,
    scratch_types=[
        pltpu.VMEM((NCH // NC, C), jnp.int32),
        pltpu.VMEM((C, 16), _f32),
        pltpu.VMEM_SHARED((N_PAD, 16), _f32),
    ] + [pltpu.SemaphoreType.DMA] * KB,
    compiler_params=_sc_params,
)


_RB = 1024
_GRID = N_PAD // _RB


def _norm_body(hist_ref, disb_ref, invcb_ref):
    cnt = hist_ref[0, :, 0:1] + hist_ref[1, :, 0:1]
    disb_ref[...] = jnp.broadcast_to(lax.rsqrt(cnt + 1.0), (_RB, 16))
    invcb_ref[...] = jnp.broadcast_to(1.0 / jnp.maximum(cnt, 1.0), (_RB, 16))


_norm_kernel = pl.pallas_call(
    _norm_body,
    grid=(_GRID,),
    in_specs=[pl.BlockSpec((NC, _RB, 16), lambda i: (0, i, 0))],
    out_specs=[pl.BlockSpec((_RB, 16), lambda i: (i, 0))] * 2,
    out_shape=[jax.ShapeDtypeStruct((N_PAD, 16), _f32)] * 2,
)


def _make_round(mode):
    def body(*refs):
        it = iter(refs)
        if mode == "prop1":
            x_hbm = next(it)
        else:
            g_hbm = next(it)
        rc_hbm = next(it)
        scb_hbm = next(it)
        if mode == "prop1":
            aux_hbm = x_hbm
        elif mode == "prop2":
            aux_hbm = next(it)
        else:
            aux_hbm = next(it)
        if mode == "prop1":
            out1_hbm = next(it)
            out2_hbm = next(it)
        elif mode == "prop2":
            out1_hbm = next(it)
        else:
            out1_hbm = next(it)
        ia_v = next(it)
        ib_v = next(it)
        abufs = tuple(next(it) for _ in range(KR))
        bbufs = tuple(next(it) for _ in range(KR))
        a0 = abufs[0]
        b0 = bbufs[0]
        scb_v = next(it)
        acc_sh = next(it)
        g_sh = next(it)
        gsa, ssa, gsb, ssb, isa, isb = (next(it) for _ in range(6))

        c = lax.axis_index("c")
        s = lax.axis_index("s")
        pltpu.sync_copy(scb_hbm.at[pl.ds(s * RPT, RPT)], scb_v)

        if mode == "prop1":
            def initc(t, carry):
                lrow = s * RPT + t * C
                grow = c * N_PAD + lrow
                pltpu.sync_copy(x_hbm.at[pl.ds(grow, C)], a0)

                def rowf(r, carry2):
                    dv = scb_v[t * C + r, :]
                    for q in range(H // 16):
                        sl = pl.ds(q * 16, 16)
                        a0[r, sl] = dv * a0[r, sl]
                    return carry2

                lax.fori_loop(0, C, rowf, 0)
                pltpu.sync_copy(a0, g_sh.at[pl.ds(lrow, C)])
                pltpu.sync_copy(a0, acc_sh.at[pl.ds(lrow, C)])
                return carry

            lax.fori_loop(0, TPC, initc, 0)
        elif mode == "prop2":
            def initc(t, carry):
                lrow = s * RPT + t * C
                pltpu.sync_copy(g_hbm.at[pl.ds(c * N_PAD + lrow, C)], a0)
                pltpu.sync_copy(a0, g_sh.at[pl.ds(lrow, C)])
                pltpu.sync_copy(a0, acc_sh.at[pl.ds(lrow, C)])
                return carry

            lax.fori_loop(0, TPC, initc, 0)
        else:
            def initc(t, carry):
                lrow = s * RPT + t * C
                pltpu.sync_copy(g_hbm.at[pl.ds(c * N_PAD + lrow, C)], a0)
                pltpu.sync_copy(a0, g_sh.at[pl.ds(lrow, C)])
                return carry

            lax.fori_loop(0, TPC, initc, 0)

            def zrow(r, carry):
                for q in range(H // 16):
                    a0[r, pl.ds(q * 16, 16)] = jnp.zeros((16,), _f32)
                return carry

            lax.fori_loop(0, C, zrow, 0)

            def zinit(t, carry):
                pltpu.sync_copy(a0, acc_sh.at[pl.ds(s * RPT + t * C, C)])
                return carry

            lax.fori_loop(0, TPC, zinit, 0)
        plsc.subcore_barrier()

        def gathers(bufs, idxv, sem):
            return [
                pltpu.async_copy(g_sh.at[idxv.at[j]], bufs[j], sem)
                for j in range(KR)
            ]

        def scatters(bufs, idxv, sem):
            return [
                pltpu.async_copy(bufs[j], acc_sh.at[idxv.at[KR + j]],
                                 sem, add=True)
                for j in range(KR)
            ]

        def prefetch(p, idxv, sem):
            return pltpu.async_copy(rc_hbm.at[c, s, p], idxv, sem)

        def wait_all(descs):
            for d in descs:
                d.wait()

        pltpu.sync_copy(rc_hbm.at[c, s, 0], ia_v)
        wait_all(gathers(abufs, ia_v, gsa))
        pltpu.sync_copy(rc_hbm.at[c, s, 1], ib_v)

        def pair(q, carry):
            pa = 2 * q
            pb = 2 * q + 1
            sa = scatters(abufs, ia_v, ssa)
            gb = gathers(bbufs, ib_v, gsb)
            wait_all(sa)
            ia = prefetch(pa + 2, ia_v, isa)
            wait_all(gb)
            sb = scatters(bbufs, ib_v, ssb)
            ia.wait()
            ga = gathers(abufs, ia_v, gsa)
            wait_all(sb)
            ib = prefetch(pb + 2, ib_v, isb)
            wait_all(ga)
            ib.wait()
            return carry

        lax.fori_loop(0, NPH // 2 - 1, pair, 0)
        sa = scatters(abufs, ia_v, ssa)
        gb = gathers(bbufs, ib_v, gsb)
        wait_all(sa)
        wait_all(gb)
        wait_all(scatters(bbufs, ib_v, ssb))
        plsc.subcore_barrier()

        def outc(t, carry):
            lrow = s * RPT + t * C
            grow = c * N_PAD + lrow
            pltpu.sync_copy(acc_sh.at[pl.ds(lrow, C)], a0)
            pltpu.sync_copy(aux_hbm.at[pl.ds(grow, C)], b0)

            def rowf(r, carry2):
                dv = scb_v[t * C + r, :]
                for q in range(H // 16):
                    sl = pl.ds(q * 16, 16)
                    v = a0[r, sl]
                    xv = b0[r, sl]
                    if mode == "prop1":
                        h = dv * v
                        a0[r, sl] = dv * h
                        b0[r, sl] = ALPHA * xv + CK * h
                    elif mode == "prop2":
                        a0[r, sl] = jnp.maximum(xv + CK * (dv * v), 0.0)
                    else:
                        a0[r, sl] = jnp.maximum(dv * v + xv, 0.0)
                return carry2

            lax.fori_loop(0, C, rowf, 0)
            if mode == "prop1":
                pltpu.sync_copy(a0, out1_hbm.at[pl.ds(grow, C)])
                pltpu.sync_copy(b0, out2_hbm.at[pl.ds(grow, C)])
            elif mode == "prop2":
                pltpu.sync_copy(a0, out1_hbm.at[pl.ds(grow, C)])
            else:
                pltpu.sync_copy(a0, out1_hbm.at[c, pl.ds(lrow, C)])
            return carry

        lax.fori_loop(0, TPC, outc, 0)

    flat_shape = jax.ShapeDtypeStruct((NC * N_PAD, H), _f32)
    if mode == "prop1":
        out_type = [flat_shape, flat_shape]
    elif mode == "prop2":
        out_type = [flat_shape]
    else:
        out_type = [jax.ShapeDtypeStruct((NC, N_PAD, H), _f32)]

    return pl.kernel(
        body,
        out_type=out_type,
        mesh=_mesh,
        scratch_types=[
            pltpu.VMEM((2 * KR, C), jnp.int32),
            pltpu.VMEM((2 * KR, C), jnp.int32),
        ] + [pltpu.VMEM((C, H), _f32)] * (2 * KR) + [
            pltpu.VMEM((RPT, 16), _f32),
            pltpu.VMEM_SHARED((N_PAD, H), _f32),
            pltpu.VMEM_SHARED((N_PAD, H), _f32),
        ] + [pltpu.SemaphoreType.DMA] * 6,
        compiler_params=_sc_params,
    )


_prop1_kernel = _make_round("prop1")
_prop2_kernel = _make_round("prop2")
_mean_kernel = _make_round("mean")


@jax.jit
def kernel(x, edge_index):
    row = edge_index[0]
    col = edge_index[1]
    fill = (jnp.arange(E_PAD - E, dtype=jnp.int32) % (N_PAD - N)) + N
    row_flat = jnp.concatenate([row, fill])
    col_flat = jnp.concatenate([col, fill])
    row_p = jnp.broadcast_to(
        row_flat.reshape(1, NS, NPH, KR, C), (NC, NS, NPH, KR, C))
    col_p = jnp.broadcast_to(
        col_flat.reshape(1, NS, NPH, KR, C), (NC, NS, NPH, KR, C))
    rc_p = jnp.concatenate([row_p, col_p], axis=3)
    x_p = jnp.pad(x, ((0, N_PAD - N), (0, 0)))
    x2 = jnp.stack([x_p[:, :H], x_p[:, H:]]).reshape(NC * N_PAD, H)
    z16 = jnp.zeros((N_PAD, 16), _f32)

    hist = _hist_kernel(col_flat.reshape(NC * NS, NCH // NC, C), z16)
    disb, invcb = _norm_kernel(hist)
    g1, o1 = _prop1_kernel(x2, rc_p, disb)
    (r,) = _prop2_kernel(g1, rc_p, disb, o1)
    (fin,) = _mean_kernel(r, rc_p, invcb, x2)
    return jnp.concatenate([fin[0, :N], fin[1, :N]], axis=1)


# speedup vs baseline: 1.3724x; 1.3724x over previous
import jax
import jax.numpy as jnp
from jax import lax
from jax.experimental import pallas as pl
from jax.experimental.pallas import tpu as pltpu
from jax.experimental.pallas import tpu_sc as plsc

N = 10000
D = 128
H = D // 2
E = 320000
ALPHA = 0.1
CK = 0.45

NC = 2
NS = 16
C = 128
KB = 2
KR = 4
NCH = 160
NPH = NCH // KR
E_PAD = NS * NCH * C
N_PAD = 10240
RPT = N_PAD // NS
TPC = RPT // C

_f32 = jnp.float32
_mesh = plsc.VectorSubcoreMesh(core_axis_name="c", subcore_axis_name="s")
_sc_params = pltpu.CompilerParams(use_tc_tiling_on_sc=False)


def _hist_body(coli_hbm, z16_hbm, out_hbm, colidx_v, ones_v, acc_sh, *sems):
    c = lax.axis_index("c")
    s = lax.axis_index("s")
    w = c * NS + s
    pltpu.sync_copy(coli_hbm.at[w], colidx_v)

    def fill(i, carry):
        ones_v[i, :] = jnp.ones((16,), _f32)
        return carry

    lax.fori_loop(0, C, fill, 0)
    pltpu.sync_copy(z16_hbm.at[pl.ds(s * RPT, RPT)],
                    acc_sh.at[pl.ds(s * RPT, RPT)])
    plsc.subcore_barrier()

    def phase(p, carry):
        descs = [
            pltpu.async_copy(ones_v, acc_sh.at[colidx_v.at[p * KB + j]],
                             sems[j], add=True)
            for j in range(KB)
        ]
        for d in descs:
            d.wait()
        return carry

    lax.fori_loop(0, (NCH // NC) // KB, phase, 0)
    plsc.subcore_barrier()
    pltpu.sync_copy(acc_sh.at[pl.ds(s * RPT, RPT)],
                    out_hbm.at[c, pl.ds(s * RPT, RPT)])


_hist_kernel = pl.kernel(
    _hist_body,
    out_type=jax.ShapeDtypeStruct((NC, N_PAD, 16), _f32),
    mesh=_mesh,
    scratch_types=[
        pltpu.VMEM((NCH // NC, C), jnp.int32),
        pltpu.VMEM((C, 16), _f32),
        pltpu.VMEM_SHARED((N_PAD, 16), _f32),
    ] + [pltpu.SemaphoreType.DMA] * KB,
    compiler_params=_sc_params,
)


_RB = 1024
_GRID = N_PAD // _RB


def _norm_body(hist_ref, disb_ref, invcb_ref):
    cnt = hist_ref[0, :, 0:1] + hist_ref[1, :, 0:1]
    disb_ref[...] = jnp.broadcast_to(lax.rsqrt(cnt + 1.0), (_RB, 16))
    invcb_ref[...] = jnp.broadcast_to(1.0 / jnp.maximum(cnt, 1.0), (_RB, 16))


_norm_kernel = pl.pallas_call(
    _norm_body,
    grid=(_GRID,),
    in_specs=[pl.BlockSpec((NC, _RB, 16), lambda i: (0, i, 0))],
    out_specs=[pl.BlockSpec((_RB, 16), lambda i: (i, 0))] * 2,
    out_shape=[jax.ShapeDtypeStruct((N_PAD, 16), _f32)] * 2,
)


def _mega_body(x_hbm, rc_hbm, disb_hbm, invcb_hbm,
               gsrc_hbm, o1_hbm, fin_hbm,
               ia_v, ib_v, a0, a1, a2, a3, b0, b1, b2, b3,
               disb_v, invcb_v, acc_sh, gsa, ssa, gsb, ssb, isa, isb):
    abufs = (a0, a1, a2, a3)
    bbufs = (b0, b1, b2, b3)
    c = lax.axis_index("c")
    s = lax.axis_index("s")
    pltpu.sync_copy(disb_hbm.at[pl.ds(s * RPT, RPT)], disb_v)
    pltpu.sync_copy(invcb_hbm.at[pl.ds(s * RPT, RPT)], invcb_v)

    def gathers(bufs, idxv, sem):
        return [
            pltpu.async_copy(gsrc_hbm.at[idxv.at[j]], bufs[j], sem)
            for j in range(KR)
        ]

    def scatters(bufs, idxv, sem):
        return [
            pltpu.async_copy(bufs[j], acc_sh.at[idxv.at[KR + j]],
                             sem, add=True)
            for j in range(KR)
        ]

    def wait_all(descs):
        for d in descs:
            d.wait()

    def scatter_loop():
        pltpu.sync_copy(rc_hbm.at[c, s, 0], ia_v)
        wait_all(gathers(abufs, ia_v, gsa))
        pltpu.sync_copy(rc_hbm.at[c, s, 1], ib_v)

        def pair(q, carry):
            pa = 2 * q
            pb = 2 * q + 1
            sa = scatters(abufs, ia_v, ssa)
            gb = gathers(bbufs, ib_v, gsb)
            wait_all(sa)
            ia = pltpu.async_copy(rc_hbm.at[c, s, pa + 2], ia_v, isa)
            wait_all(gb)
            sb = scatters(bbufs, ib_v, ssb)
            ia.wait()
            ga = gathers(abufs, ia_v, gsa)
            wait_all(sb)
            ib = pltpu.async_copy(rc_hbm.at[c, s, pb + 2], ib_v, isb)
            wait_all(ga)
            ib.wait()
            return carry

        lax.fori_loop(0, NPH // 2 - 1, pair, 0)
        sa = scatters(abufs, ia_v, ssa)
        gb = gathers(bbufs, ib_v, gsb)
        wait_all(sa)
        wait_all(gb)
        wait_all(scatters(bbufs, ib_v, ssb))

    def initc(t, carry):
        lrow = s * RPT + t * C
        grow = c * N_PAD + lrow
        pltpu.sync_copy(x_hbm.at[pl.ds(grow, C)], a0)

        def rowf(r, carry2):
            dv = disb_v[t * C + r, :]
            for q in range(H // 16):
                sl = pl.ds(q * 16, 16)
                a0[r, sl] = dv * a0[r, sl]
            return carry2

        lax.fori_loop(0, C, rowf, 0)
        pltpu.sync_copy(a0, gsrc_hbm.at[pl.ds(grow, C)])
        pltpu.sync_copy(a0, acc_sh.at[pl.ds(lrow, C)])
        return carry

    lax.fori_loop(0, TPC, initc, 0)
    plsc.subcore_barrier()
    scatter_loop()
    plsc.subcore_barrier()

    def tr1(t, carry):
        lrow = s * RPT + t * C
        grow = c * N_PAD + lrow
        pltpu.sync_copy(acc_sh.at[pl.ds(lrow, C)], a0)
        pltpu.sync_copy(x_hbm.at[pl.ds(grow, C)], b0)

        def rowf(r, carry2):
            dv = disb_v[t * C + r, :]
            for q in range(H // 16):
                sl = pl.ds(q * 16, 16)
                h = dv * a0[r, sl]
                a0[r, sl] = dv * h
                b0[r, sl] = ALPHA * b0[r, sl] + CK * h
            return carry2

        lax.fori_loop(0, C, rowf, 0)
        pltpu.sync_copy(a0, gsrc_hbm.at[pl.ds(grow, C)])
        pltpu.sync_copy(a0, acc_sh.at[pl.ds(lrow, C)])
        pltpu.sync_copy(b0, o1_hbm.at[pl.ds(grow, C)])
        return carry

    lax.fori_loop(0, TPC, tr1, 0)
    plsc.subcore_barrier()
    scatter_loop()
    plsc.subcore_barrier()

    def tr2(t, carry):
        lrow = s * RPT + t * C
        grow = c * N_PAD + lrow
        pltpu.sync_copy(acc_sh.at[pl.ds(lrow, C)], a0)
        pltpu.sync_copy(o1_hbm.at[pl.ds(grow, C)], b0)

        def rowf(r, carry2):
            dv = disb_v[t * C + r, :]
            for q in range(H // 16):
                sl = pl.ds(q * 16, 16)
                a0[r, sl] = jnp.maximum(b0[r, sl] + CK * (dv * a0[r, sl]),
                                        0.0)
                b0[r, sl] = jnp.zeros((16,), _f32)
            return carry2

        lax.fori_loop(0, C, rowf, 0)
        pltpu.sync_copy(a0, gsrc_hbm.at[pl.ds(grow, C)])
        pltpu.sync_copy(b0, acc_sh.at[pl.ds(lrow, C)])
        return carry

    lax.fori_loop(0, TPC, tr2, 0)
    plsc.subcore_barrier()
    scatter_loop()
    plsc.subcore_barrier()

    def tr3(t, carry):
        lrow = s * RPT + t * C
        grow = c * N_PAD + lrow
        pltpu.sync_copy(acc_sh.at[pl.ds(lrow, C)], a0)
        pltpu.sync_copy(x_hbm.at[pl.ds(grow, C)], b0)

        def rowf(r, carry2):
            iv = invcb_v[t * C + r, :]
            for q in range(H // 16):
                sl = pl.ds(q * 16, 16)
                a0[r, sl] = jnp.maximum(iv * a0[r, sl] + b0[r, sl], 0.0)
            return carry2

        lax.fori_loop(0, C, rowf, 0)
        pltpu.sync_copy(a0, fin_hbm.at[c, pl.ds(lrow, C)])
        return carry

    lax.fori_loop(0, TPC, tr3, 0)


_flat_shape = jax.ShapeDtypeStruct((NC * N_PAD, H), _f32)
_mega_kernel = pl.kernel(
    _mega_body,
    out_type=[_flat_shape, _flat_shape,
              jax.ShapeDtypeStruct((NC, N_PAD, H), _f32)],
    mesh=_mesh,
    scratch_types=[
        pltpu.VMEM((2 * KR, C), jnp.int32),
        pltpu.VMEM((2 * KR, C), jnp.int32),
    ] + [pltpu.VMEM((C, H), _f32)] * (2 * KR) + [
        pltpu.VMEM((RPT, 16), _f32),
        pltpu.VMEM((RPT, 16), _f32),
        pltpu.VMEM_SHARED((N_PAD, H), _f32),
    ] + [pltpu.SemaphoreType.DMA] * 6,
    compiler_params=_sc_params,
)


@jax.jit
def kernel(x, edge_index):
    row = edge_index[0]
    col = edge_index[1]
    fill = (jnp.arange(E_PAD - E, dtype=jnp.int32) % (N_PAD - N)) + N
    row_flat = jnp.concatenate([row, fill])
    col_flat = jnp.concatenate([col, fill])
    row_p = jnp.stack([row_flat, row_flat + N_PAD]).reshape(NC, NS, NPH, KR, C)
    col_p = jnp.broadcast_to(
        col_flat.reshape(1, NS, NPH, KR, C), (NC, NS, NPH, KR, C))
    rc_p = jnp.concatenate([row_p, col_p], axis=3)
    x_p = jnp.pad(x, ((0, N_PAD - N), (0, 0)))
    x2 = jnp.stack([x_p[:, :H], x_p[:, H:]]).reshape(NC * N_PAD, H)
    z16 = jnp.zeros((N_PAD, 16), _f32)

    hist = _hist_kernel(col_flat.reshape(NC * NS, NCH // NC, C), z16)
    disb, invcb = _norm_kernel(hist)
    _, _, fin = _mega_kernel(x2, rc_p, disb, invcb)
    return jnp.concatenate([fin[0, :N], fin[1, :N]], axis=1)


# speedup vs baseline: 1.4094x; 1.0270x over previous
import jax
import jax.numpy as jnp
from jax import lax
from jax.experimental import pallas as pl
from jax.experimental.pallas import tpu as pltpu
from jax.experimental.pallas import tpu_sc as plsc

N = 10000
D = 128
H = D // 2
E = 320000
ALPHA = 0.1
CK = 0.45

NC = 2
NS = 16
C = 128
KB = 2
KR = 4
NCH = 160
NPH = NCH // KR
E_PAD = NS * NCH * C
N_PAD = 10240
RPT = N_PAD // NS
TPC = RPT // C

_f32 = jnp.float32
_mesh = plsc.VectorSubcoreMesh(core_axis_name="c", subcore_axis_name="s")
_sc_params = pltpu.CompilerParams(use_tc_tiling_on_sc=False,
                                  needs_layout_passes=False)


def _hist_body(coli_hbm, z16_hbm, out_hbm, colidx_v, ones_v, acc_sh, *sems):
    c = lax.axis_index("c")
    s = lax.axis_index("s")
    w = c * NS + s
    pltpu.sync_copy(coli_hbm.at[w], colidx_v)

    def fill(i, carry):
        ones_v[i, :] = jnp.ones((16,), _f32)
        return carry

    lax.fori_loop(0, C, fill, 0)
    pltpu.sync_copy(z16_hbm.at[pl.ds(s * RPT, RPT)],
                    acc_sh.at[pl.ds(s * RPT, RPT)])
    plsc.subcore_barrier()

    def phase(p, carry):
        descs = [
            pltpu.async_copy(ones_v, acc_sh.at[colidx_v.at[p * KB + j]],
                             sems[j], add=True)
            for j in range(KB)
        ]
        for d in descs:
            d.wait()
        return carry

    lax.fori_loop(0, (NCH // NC) // KB, phase, 0)
    plsc.subcore_barrier()
    pltpu.sync_copy(acc_sh.at[pl.ds(s * RPT, RPT)],
                    out_hbm.at[c, pl.ds(s * RPT, RPT)])


_hist_kernel = pl.kernel(
    _hist_body,
    out_type=jax.ShapeDtypeStruct((NC, N_PAD, 16), _f32),
    mesh=_mesh,
    scratch_types=[
        pltpu.VMEM((NCH // NC, C), jnp.int32),
        pltpu.VMEM((C, 16), _f32),
        pltpu.VMEM_SHARED((N_PAD, 16), _f32),
    ] + [pltpu.SemaphoreType.DMA] * KB,
    compiler_params=_sc_params,
)


def _mega_body(x_hbm, rc_hbm, hist_hbm,
               gsrc_hbm, o1_hbm, fin_hbm,
               ia_v, ib_v, a0, a1, a2, a3, b0, b1, b2, b3,
               disb_v, invcb_v, acc_sh, gsa, ssa, gsb, ssb, isa, isb):
    abufs = (a0, a1, a2, a3)
    bbufs = (b0, b1, b2, b3)
    c = lax.axis_index("c")
    s = lax.axis_index("s")

    pltpu.sync_copy(hist_hbm.at[0, pl.ds(s * RPT, RPT)], disb_v)
    pltpu.sync_copy(hist_hbm.at[1, pl.ds(s * RPT, RPT)], invcb_v)

    def normf(i, carry):
        cnt = disb_v[i, :] + invcb_v[i, :]
        y = cnt + 1.0
        iv = lax.sub(jnp.full((16,), 0x5F3759DF, jnp.int32),
                     lax.shift_right_logical(
                         plsc.bitcast(y, jnp.int32),
                         jnp.ones((16,), jnp.int32)))
        seed = plsc.bitcast(iv, _f32)
        for _ in range(3):
            seed = seed * (1.5 - 0.5 * y * seed * seed)
        disb_v[i, :] = seed
        invcb_v[i, :] = 1.0 / jnp.maximum(cnt, 1.0)
        return carry

    lax.fori_loop(0, RPT, normf, 0)

    def gathers(bufs, idxv, sem):
        return [
            pltpu.async_copy(gsrc_hbm.at[idxv.at[j]], bufs[j], sem)
            for j in range(KR)
        ]

    def scatters(bufs, idxv, sem):
        return [
            pltpu.async_copy(bufs[j], acc_sh.at[idxv.at[KR + j]],
                             sem, add=True)
            for j in range(KR)
        ]

    def wait_all(descs):
        for d in descs:
            d.wait()

    def scatter_loop():
        pltpu.sync_copy(rc_hbm.at[c, s, 0], ia_v)
        wait_all(gathers(abufs, ia_v, gsa))
        pltpu.sync_copy(rc_hbm.at[c, s, 1], ib_v)

        def pair(q, carry):
            pa = 2 * q
            pb = 2 * q + 1
            sa = scatters(abufs, ia_v, ssa)
            gb = gathers(bbufs, ib_v, gsb)
            wait_all(sa)
            ia = pltpu.async_copy(rc_hbm.at[c, s, pa + 2], ia_v, isa)
            wait_all(gb)
            sb = scatters(bbufs, ib_v, ssb)
            ia.wait()
            ga = gathers(abufs, ia_v, gsa)
            wait_all(sb)
            ib = pltpu.async_copy(rc_hbm.at[c, s, pb + 2], ib_v, isb)
            wait_all(ga)
            ib.wait()
            return carry

        lax.fori_loop(0, NPH // 2 - 1, pair, 0)
        sa = scatters(abufs, ia_v, ssa)
        gb = gathers(bbufs, ib_v, gsb)
        wait_all(sa)
        wait_all(gb)
        wait_all(scatters(bbufs, ib_v, ssb))

    def initc(t, carry):
        lrow = s * RPT + t * C
        grow = c * N_PAD + lrow
        pltpu.sync_copy(x_hbm.at[pl.ds(grow, C)], a0)

        def rowf(r, carry2):
            dv = disb_v[t * C + r, :]
            for q in range(H // 16):
                sl = pl.ds(q * 16, 16)
                a0[r, sl] = dv * a0[r, sl]
            return carry2

        lax.fori_loop(0, C, rowf, 0)
        pltpu.sync_copy(a0, gsrc_hbm.at[pl.ds(grow, C)])
        pltpu.sync_copy(a0, acc_sh.at[pl.ds(lrow, C)])
        return carry

    lax.fori_loop(0, TPC, initc, 0)
    plsc.subcore_barrier()
    scatter_loop()
    plsc.subcore_barrier()

    def tr1(t, carry):
        lrow = s * RPT + t * C
        grow = c * N_PAD + lrow
        pltpu.sync_copy(acc_sh.at[pl.ds(lrow, C)], a0)
        pltpu.sync_copy(x_hbm.at[pl.ds(grow, C)], b0)

        def rowf(r, carry2):
            dv = disb_v[t * C + r, :]
            for q in range(H // 16):
                sl = pl.ds(q * 16, 16)
                h = dv * a0[r, sl]
                a0[r, sl] = dv * h
                b0[r, sl] = ALPHA * b0[r, sl] + CK * h
            return carry2

        lax.fori_loop(0, C, rowf, 0)
        pltpu.sync_copy(a0, gsrc_hbm.at[pl.ds(grow, C)])
        pltpu.sync_copy(a0, acc_sh.at[pl.ds(lrow, C)])
        pltpu.sync_copy(b0, o1_hbm.at[pl.ds(grow, C)])
        return carry

    lax.fori_loop(0, TPC, tr1, 0)
    plsc.subcore_barrier()
    scatter_loop()
    plsc.subcore_barrier()

    def tr2(t, carry):
        lrow = s * RPT + t * C
        grow = c * N_PAD + lrow
        pltpu.sync_copy(acc_sh.at[pl.ds(lrow, C)], a0)
        pltpu.sync_copy(o1_hbm.at[pl.ds(grow, C)], b0)

        def rowf(r, carry2):
            dv = disb_v[t * C + r, :]
            for q in range(H // 16):
                sl = pl.ds(q * 16, 16)
                a0[r, sl] = jnp.maximum(b0[r, sl] + CK * (dv * a0[r, sl]),
                                        0.0)
                b0[r, sl] = jnp.zeros((16,), _f32)
            return carry2

        lax.fori_loop(0, C, rowf, 0)
        pltpu.sync_copy(a0, gsrc_hbm.at[pl.ds(grow, C)])
        pltpu.sync_copy(b0, acc_sh.at[pl.ds(lrow, C)])
        return carry

    lax.fori_loop(0, TPC, tr2, 0)
    plsc.subcore_barrier()
    scatter_loop()
    plsc.subcore_barrier()

    def tr3(t, carry):
        lrow = s * RPT + t * C
        grow = c * N_PAD + lrow
        pltpu.sync_copy(acc_sh.at[pl.ds(lrow, C)], a0)
        pltpu.sync_copy(x_hbm.at[pl.ds(grow, C)], b0)

        def rowf(r, carry2):
            iv = invcb_v[t * C + r, :]
            for q in range(H // 16):
                sl = pl.ds(q * 16, 16)
                a0[r, sl] = jnp.maximum(iv * a0[r, sl] + b0[r, sl], 0.0)
            return carry2

        lax.fori_loop(0, C, rowf, 0)
        pltpu.sync_copy(a0, fin_hbm.at[c, pl.ds(lrow, C)])
        return carry

    lax.fori_loop(0, TPC, tr3, 0)


_flat_shape = jax.ShapeDtypeStruct((NC * N_PAD, H), _f32)
_mega_kernel = pl.kernel(
    _mega_body,
    out_type=[_flat_shape, _flat_shape,
              jax.ShapeDtypeStruct((NC, N_PAD, H), _f32)],
    mesh=_mesh,
    scratch_types=[
        pltpu.VMEM((2 * KR, C), jnp.int32),
        pltpu.VMEM((2 * KR, C), jnp.int32),
    ] + [pltpu.VMEM((C, H), _f32)] * (2 * KR) + [
        pltpu.VMEM((RPT, 16), _f32),
        pltpu.VMEM((RPT, 16), _f32),
        pltpu.VMEM_SHARED((N_PAD, H), _f32),
    ] + [pltpu.SemaphoreType.DMA] * 6,
    compiler_params=_sc_params,
)


@jax.jit
def kernel(x, edge_index):
    row = edge_index[0]
    col = edge_index[1]
    fill = (jnp.arange(E_PAD - E, dtype=jnp.int32) % (N_PAD - N)) + N
    row_flat = jnp.concatenate([row, fill])
    col_flat = jnp.concatenate([col, fill])
    row_p = jnp.stack([row_flat, row_flat + N_PAD]).reshape(NC, NS, NPH, KR, C)
    col_p = jnp.broadcast_to(
        col_flat.reshape(1, NS, NPH, KR, C), (NC, NS, NPH, KR, C))
    rc_p = jnp.concatenate([row_p, col_p], axis=3)
    x_p = jnp.pad(x, ((0, N_PAD - N), (0, 0)))
    x2 = jnp.stack([x_p[:, :H], x_p[:, H:]]).reshape(NC * N_PAD, H)
    z16 = jnp.zeros((N_PAD, 16), _f32)

    hist = _hist_kernel(col_flat.reshape(NC * NS, NCH // NC, C), z16)
    _, _, fin = _mega_kernel(x2, rc_p, hist)
    return jnp.concatenate([fin[0, :N], fin[1, :N]], axis=1)
